# baseline (speedup 1.0000x reference)
import functools

import jax
import jax.numpy as jnp
from jax import lax
from jax.experimental import pallas as pl
from jax.experimental.pallas import tpu as pltpu
from jax.experimental.pallas import tpu_sc as plsc

_NC = 2
_NS = 16
_NW = _NC * _NS
_L = 16
_R = 4
_IDX_BITS = 13
_HMASK = -65536


def _bf16_bits(v):
    return lax.bitcast_convert_type(v, jnp.int32) + 0x8000


def _prep_body(wt_ref, ia_ref, ib_ref, c01_ref, c23_ref, pk_ref):
    wt = wt_ref[...]
    m = jnp.max(wt, axis=0, keepdims=True)
    e = jnp.exp(wt - m)
    s = e / jnp.sum(e, axis=0, keepdims=True)
    r = [s[i : i + 1, :] for i in range(14)]
    c0 = r[7] + r[8] + r[9] + r[10] + r[11] + r[12] + r[13]
    c1 = r[1] + r[2] + r[5] + r[6] - r[7] - r[8] - r[11] - r[12]
    c2 = r[3] + r[4] + r[5] + r[6] - r[7] - r[8] - r[9] - r[10]
    c3 = (r[0] - r[1] - r[3] - 2.0 * r[5] - r[6] + r[7] + 2.0 * r[8]
          + r[10] + r[12] - r[13])
    c01_ref[...] = (
        ((_bf16_bits(c0) >> 16) & 0xFFFF) | (_bf16_bits(c1) & _HMASK))
    c23_ref[...] = (
        ((_bf16_bits(c2) >> 16) & 0xFFFF) | (_bf16_bits(c3) & _HMASK))
    pk_ref[...] = ia_ref[...] | (ib_ref[...] << _IDX_BITS)


def _prep(weights, idx_a, idx_b):
    out_dim = weights.shape[0]
    shp = jax.ShapeDtypeStruct((1, out_dim), jnp.int32)
    return pl.pallas_call(
        _prep_body,
        out_shape=[shp, shp, shp],
    )(weights.T, idx_a[None, :], idx_b[None, :])


def _make_sc_kernel(batch, in_dim, out_dim):
    rows_per_w = batch // _NW
    ngroups = rows_per_w // _R
    half = out_dim // 2
    hchunks = half // _L
    mesh = plsc.VectorSubcoreMesh(
        core_axis_name="c", subcore_axis_name="s",
        num_cores=_NC, num_subcores=_NS)

    @functools.partial(
        pl.kernel,
        mesh=mesh,
        compiler_params=pltpu.CompilerParams(needs_layout_passes=False),
        out_type=jax.ShapeDtypeStruct((batch, out_dim), jnp.float32),
        scratch_types=[
            pltpu.VMEM((out_dim,), jnp.int32),
            pltpu.VMEM((out_dim,), jnp.int32),
            pltpu.VMEM((out_dim,), jnp.int32),
            pltpu.VMEM((_R, in_dim), jnp.float32),
            pltpu.VMEM((_R, in_dim), jnp.float32),
            pltpu.VMEM((_R, half), jnp.float32),
            pltpu.VMEM((_R, half), jnp.float32),
            pltpu.SemaphoreType.DMA,
            pltpu.SemaphoreType.DMA,
            pltpu.SemaphoreType.DMA,
            pltpu.SemaphoreType.DMA,
        ],
    )
    def sc_kernel(x_hbm, c01_hbm, c23_hbm, pk_hbm, out_hbm,
                  pk_v, c01_v, c23_v, x0_v, x1_v, oh0_v, oh1_v,
                  xs0, xs1, os0, os1):
        wid = lax.axis_index("s") * _NC + lax.axis_index("c")
        base = wid * rows_per_w
        pltpu.sync_copy(pk_hbm, pk_v)
        pltpu.sync_copy(c01_hbm, c01_v)
        pltpu.sync_copy(c23_hbm, c23_v)

        xbufs = (x0_v, x1_v)
        xsems = (xs0, xs1)
        obufs = (oh0_v, oh1_v)
        osems = (os0, os1)

        def x_dma(g, phase):
            return pltpu.make_async_copy(
                x_hbm.at[pl.ds(base + g * _R, _R)], xbufs[phase], xsems[phase])

        def o_dma(g, h):
            return pltpu.make_async_copy(
                obufs[h],
                out_hbm.at[pl.ds(base + g * _R, _R), pl.ds(h * half, half)],
                osems[h])

        def unpack_pair(v):
            lo = plsc.bitcast(v << 16, jnp.float32)
            hi = plsc.bitcast(lax.bitwise_and(v, _HMASK), jnp.float32)
            return lo, hi

        x_dma(0, 0).start()

        def outer(i, carry):
            go = i * 2
            for phase in range(2):
                g = go + phase
                xbuf = xbufs[phase]
                x_dma(g, phase).wait()

                @pl.when(g + 1 < ngroups)
                def _():
                    x_dma(g + 1, 1 - phase).start()

                for h in range(2):
                    obuf = obufs[h]

                    @pl.when(g >= 1)
                    def _():
                        o_dma(g - 1, h).wait()

                    @plsc.parallel_loop(0, hchunks, unroll=8)
                    def chunk(jc):
                        offl = jc * _L
                        offg = h * half + offl
                        pv = pk_v[pl.ds(offg, _L)]
                        ja = lax.bitwise_and(pv, (1 << _IDX_BITS) - 1)
                        jb = lax.shift_right_logical(pv, _IDX_BITS)
                        c0, c1 = unpack_pair(c01_v[pl.ds(offg, _L)])
                        c2, c3 = unpack_pair(c23_v[pl.ds(offg, _L)])
                        for r in range(_R):
                            rv = jnp.full((_L,), r, jnp.int32)
                            a = plsc.load_gather(xbuf, [rv, ja])
                            b = plsc.load_gather(xbuf, [rv, jb])
                            obuf[r, pl.ds(offl, _L)] = (
                                c0 + a * (c1 + c3 * b) + c2 * b)

                    o_dma(g, h).start()
            return carry

        lax.fori_loop(0, ngroups // 2, outer, 0)
        o_dma(ngroups - 1, 0).wait()
        o_dma(ngroups - 1, 1).wait()

    return sc_kernel


def kernel(x, weights, idx_a, idx_b):
    batch, in_dim = x.shape
    out_dim = weights.shape[0]
    c01, c23, pk = _prep(
        weights, idx_a.astype(jnp.int32), idx_b.astype(jnp.int32))
    sc = _make_sc_kernel(batch, in_dim, out_dim)
    return sc(x, c01.reshape(-1), c23.reshape(-1), pk.reshape(-1))


# speedup vs baseline: 1.5509x; 1.5509x over previous
import functools

import jax
import jax.numpy as jnp
from jax import lax
from jax.experimental import pallas as pl
from jax.experimental.pallas import tpu as pltpu
from jax.experimental.pallas import tpu_sc as plsc

_NC = 2
_NS = 16
_NW = _NC * _NS
_L = 16
_R = 4
_IDX_BITS = 13
_HMASK = -65536


def _bf16_bits(v):
    return lax.bitcast_convert_type(v, jnp.int32) + 0x8000


def _prep_body(wt_ref, ia_ref, ib_ref, c01_ref, c23_ref, pk_ref):
    wt = wt_ref[...]
    m = jnp.max(wt, axis=0, keepdims=True)
    e = jnp.exp(wt - m)
    s = e / jnp.sum(e, axis=0, keepdims=True)
    r = [s[i : i + 1, :] for i in range(14)]
    c0 = r[7] + r[8] + r[9] + r[10] + r[11] + r[12] + r[13]
    c1 = r[1] + r[2] + r[5] + r[6] - r[7] - r[8] - r[11] - r[12]
    c2 = r[3] + r[4] + r[5] + r[6] - r[7] - r[8] - r[9] - r[10]
    c3 = (r[0] - r[1] - r[3] - 2.0 * r[5] - r[6] + r[7] + 2.0 * r[8]
          + r[10] + r[12] - r[13])
    c01_ref[...] = (
        ((_bf16_bits(c0) >> 16) & 0xFFFF) | (_bf16_bits(c1) & _HMASK))
    c23_ref[...] = (
        ((_bf16_bits(c2) >> 16) & 0xFFFF) | (_bf16_bits(c3) & _HMASK))
    pk_ref[...] = ia_ref[...] | (ib_ref[...] << _IDX_BITS)


def _prep(weights, idx_a, idx_b):
    out_dim = weights.shape[0]
    shp = jax.ShapeDtypeStruct((1, out_dim), jnp.int32)
    return pl.pallas_call(
        _prep_body,
        out_shape=[shp, shp, shp],
    )(weights.T, idx_a[None, :], idx_b[None, :])


def _make_sc_kernel(batch, in_dim, out_dim):
    rows_per_w = batch // _NW
    ngroups = rows_per_w // _R
    half = out_dim // 2
    hchunks = half // _L
    mesh = plsc.VectorSubcoreMesh(
        core_axis_name="c", subcore_axis_name="s",
        num_cores=_NC, num_subcores=_NS)

    @functools.partial(
        pl.kernel,
        mesh=mesh,
        compiler_params=pltpu.CompilerParams(needs_layout_passes=False),
        out_type=jax.ShapeDtypeStruct((batch, out_dim), jnp.float32),
        scratch_types=[
            pltpu.VMEM((out_dim,), jnp.int32),
            pltpu.VMEM((out_dim,), jnp.int32),
            pltpu.VMEM((out_dim,), jnp.int32),
            pltpu.VMEM((_R, in_dim), jnp.float32),
            pltpu.VMEM((_R, in_dim), jnp.float32),
            pltpu.VMEM((_R, half), jnp.float32),
            pltpu.VMEM((_R, half), jnp.float32),
            pltpu.SemaphoreType.DMA,
            pltpu.SemaphoreType.DMA,
            pltpu.SemaphoreType.DMA,
            pltpu.SemaphoreType.DMA,
        ],
    )
    def sc_kernel(x_hbm, c01_hbm, c23_hbm, pk_hbm, out_hbm,
                  pk_v, c01_v, c23_v, x0_v, x1_v, oh0_v, oh1_v,
                  xs0, xs1, os0, os1):
        wid = lax.axis_index("s") * _NC + lax.axis_index("c")
        base = wid * rows_per_w
        pltpu.sync_copy(pk_hbm, pk_v)
        pltpu.sync_copy(c01_hbm, c01_v)
        pltpu.sync_copy(c23_hbm, c23_v)

        xbufs = (x0_v, x1_v)
        xsems = (xs0, xs1)
        obufs = (oh0_v, oh1_v)
        osems = (os0, os1)

        def x_dma(g, phase):
            return pltpu.make_async_copy(
                x_hbm.at[pl.ds(base + g * _R, _R)], xbufs[phase], xsems[phase])

        def o_dma(g, h):
            return pltpu.make_async_copy(
                obufs[h],
                out_hbm.at[pl.ds(base + g * _R, _R), pl.ds(h * half, half)],
                osems[h])

        def unpack_pair(v):
            lo = plsc.bitcast(v << 16, jnp.float32)
            hi = plsc.bitcast(lax.bitwise_and(v, _HMASK), jnp.float32)
            return lo, hi

        x_dma(0, 0).start()

        def outer(i, carry):
            go = i * 2
            for phase in range(2):
                g = go + phase
                xbuf = xbufs[phase]
                x_dma(g, phase).wait()

                @pl.when(g + 1 < ngroups)
                def _():
                    x_dma(g + 1, 1 - phase).start()

                for h in range(2):
                    obuf = obufs[h]

                    @pl.when(g >= 1)
                    def _():
                        o_dma(g - 1, h).wait()

                    @plsc.parallel_loop(0, hchunks, unroll=2)
                    def chunk(jc):
                        offl = jc * _L
                        offg = h * half + offl
                        pv = pk_v[pl.ds(offg, _L)]
                        ja = lax.bitwise_and(pv, (1 << _IDX_BITS) - 1)
                        jb = lax.shift_right_logical(pv, _IDX_BITS)
                        c0, c1 = unpack_pair(c01_v[pl.ds(offg, _L)])
                        c2, c3 = unpack_pair(c23_v[pl.ds(offg, _L)])
                        for r in range(_R):
                            rv = jnp.full((_L,), r, jnp.int32)
                            a = plsc.load_gather(xbuf, [rv, ja])
                            b = plsc.load_gather(xbuf, [rv, jb])
                            obuf[r, pl.ds(offl, _L)] = (
                                c0 + a * (c1 + c3 * b) + c2 * b)

                    o_dma(g, h).start()
            return carry

        lax.fori_loop(0, ngroups // 2, outer, 0)
        o_dma(ngroups - 1, 0).wait()
        o_dma(ngroups - 1, 1).wait()

    return sc_kernel


def kernel(x, weights, idx_a, idx_b):
    batch, in_dim = x.shape
    out_dim = weights.shape[0]
    c01, c23, pk = _prep(
        weights, idx_a.astype(jnp.int32), idx_b.astype(jnp.int32))
    sc = _make_sc_kernel(batch, in_dim, out_dim)
    return sc(x, c01.reshape(-1), c23.reshape(-1), pk.reshape(-1))
